# baseline (speedup 1.0000x reference)
import functools

import jax
import jax.numpy as jnp
from jax import lax
from jax.experimental import pallas as pl
from jax.experimental.pallas import tpu as pltpu
from jax.experimental.pallas import tpu_sc as plsc

_N = 10000
_E = 320000
_CH = 128
_G = 16
_NC = 2
_NS = 16
_NW = _NC * _NS
_CHUNK = 80
_CPT = _E // _NW
_NCH = _CPT // _CHUNK
_NB = 2
_ROWS_PT = 624
_ROWS_REM = _N - _ROWS_PT * _NS

_F32 = jnp.float32


def _mesh():
    return plsc.VectorSubcoreMesh(core_axis_name="c", subcore_axis_name="s")


def _stripe_copy(src_of, dst_of, s):
    pltpu.sync_copy(src_of(s * _ROWS_PT, _ROWS_PT), dst_of(s * _ROWS_PT, _ROWS_PT))

    @pl.when(s == _NS - 1)
    def _tail():
        pltpu.sync_copy(src_of(_ROWS_PT * _NS, _ROWS_REM),
                        dst_of(_ROWS_PT * _NS, _ROWS_REM))


def _sc_degree(dst32, ones_rows, zeros_pad):
    @functools.partial(
        pl.kernel,
        out_type=jax.ShapeDtypeStruct((_NC, _N, _CH), _F32),
        mesh=_mesh(),
        scratch_types=[
            pltpu.VMEM((_NCH, _CHUNK), jnp.int32),
            pltpu.VMEM((_CHUNK, _CH), _F32),
            pltpu.VMEM_SHARED((_N, _CH), _F32),
        ],
    )
    def deg_kernel(dst_hbm, ones_hbm, z_hbm, out_hbm, idx_v, ones_v, acc_sh):
        c = lax.axis_index("c")
        s = lax.axis_index("s")

        _stripe_copy(lambda b, n: z_hbm.at[pl.ds(0, n)],
                     lambda b, n: acc_sh.at[pl.ds(b, n)], s)
        pltpu.sync_copy(dst_hbm.at[s * _NC + c], idx_v)
        pltpu.sync_copy(ones_hbm, ones_v)
        plsc.subcore_barrier()

        @pl.loop(0, _NCH)
        def _scat(j):
            pltpu.sync_copy(ones_v, acc_sh.at[idx_v.at[j]], add=True)

        plsc.subcore_barrier()
        _stripe_copy(lambda b, n: acc_sh.at[pl.ds(b, n)],
                     lambda b, n: out_hbm.at[c, pl.ds(b, n)], s)

    return deg_kernel(dst32, ones_rows, zeros_pad)


def _sc_propagate(u, src_flat, dst32, zeros_pad):
    @functools.partial(
        pl.kernel,
        out_type=jax.ShapeDtypeStruct((_NC, _N, _CH), _F32),
        mesh=_mesh(),
        scratch_types=[
            pltpu.VMEM((_CPT,), jnp.int32),
            pltpu.VMEM((_NCH, _CHUNK), jnp.int32),
        ] + [pltpu.VMEM((_CHUNK, _CH), _F32) for _ in range(_NB)] + [
            pltpu.VMEM_SHARED((_N, _CH), _F32),
        ] + [pltpu.SemaphoreType.DMA for _ in range(_NB)],
    )
    def prop_kernel(u_hbm, src_hbm, dst_hbm, z_hbm, out_hbm,
                    idx_s, idx_d, *rest):
        rows = rest[:_NB]
        acc_sh = rest[_NB]
        sems = rest[_NB + 1:]
        c = lax.axis_index("c")
        s = lax.axis_index("s")
        wid = s * _NC + c

        _stripe_copy(lambda b, n: z_hbm.at[pl.ds(0, n)],
                     lambda b, n: acc_sh.at[pl.ds(b, n)], s)
        pltpu.sync_copy(src_hbm.at[wid], idx_s)
        pltpu.sync_copy(dst_hbm.at[wid], idx_d)

        def _gather(j, b):
            pltpu.async_copy(u_hbm.at[idx_s.at[pl.ds(j * _CHUNK, _CHUNK)]],
                             rows[b], sems[b])

        def _wait(b):
            pltpu.make_async_copy(u_hbm.at[idx_s.at[pl.ds(0, _CHUNK)]],
                                  rows[b], sems[b]).wait()

        for b in range(_NB):
            _gather(b, b)
        plsc.subcore_barrier()

        @pl.loop(0, _NCH // _NB)
        def _ring(r):
            for b in range(_NB):
                j = r * _NB + b
                _wait(b)
                pltpu.sync_copy(rows[b], acc_sh.at[idx_d.at[j]], add=True)

                @pl.when(j + _NB < _NCH)
                def _reissue():
                    _gather(j + _NB, b)

        for t in range(_NCH % _NB):
            _wait(t)
            pltpu.sync_copy(rows[t],
                            acc_sh.at[idx_d.at[_NCH - _NCH % _NB + t]],
                            add=True)

        plsc.subcore_barrier()
        _stripe_copy(lambda b, n: acc_sh.at[pl.ds(b, n)],
                     lambda b, n: out_hbm.at[c, pl.ds(b, n)], s)

    return prop_kernel(u, src_flat, dst32, zeros_pad)


def _dot(a, b):
    return jax.lax.dot_general(
        a, b, (((1,), (0,)), ((), ())),
        precision=jax.lax.Precision.DEFAULT, preferred_element_type=_F32)


def _bn_cols(z, gamma, beta):
    mu = jnp.mean(z, axis=0, keepdims=True)
    var = jnp.mean((z - mu) * (z - mu), axis=0, keepdims=True)
    return (z - mu) * jax.lax.rsqrt(var + 1e-5) * gamma + beta


def _tc_dinv(degp):
    def body(degp_r, dinv_ref):
        deg = 1.0 + degp_r[0, :, 0:1] + degp_r[1, :, 0:1]
        dinv_ref[...] = jax.lax.rsqrt(deg)

    return pl.pallas_call(
        body,
        out_shape=jax.ShapeDtypeStruct((_N, 1), _F32),
    )(degp)


def _tc_prologue(x, lap_pe, dinv, Wn, bn, gl, bel, Wl, bl, W0):
    def body(x_r, lap_r, dinv_r, Wn_r, bn_r, gl_r, bel_r, Wl_r, bl_r, W0_r,
             u_ref):
        xn = _dot(x_r[...], Wn_r[...]) + bn_r[...]
        lap = _bn_cols(lap_r[...], gl_r[...], bel_r[...])
        lap = _dot(lap, Wl_r[...]) + bl_r[...]
        h = jnp.concatenate([xn, lap], axis=1)
        u_ref[...] = _dot(h, W0_r[...]) * dinv_r[...]

    return pl.pallas_call(
        body,
        out_shape=jax.ShapeDtypeStruct((_N, _CH), _F32),
    )(x, lap_pe, dinv, Wn, bn, gl, bel, Wl, bl, W0)


def _tc_layer(acc_split, u, dinv, b_prev, W_next):
    def body(acc_r, u_r, dinv_r, b_r, W_r, out_ref):
        dinv = dinv_r[...]
        w = acc_r[0] + acc_r[1] + u_r[...]
        h = jnp.maximum(dinv * w + b_r[...], 0.0)
        out_ref[...] = _dot(h, W_r[...]) * dinv

    return pl.pallas_call(
        body,
        out_shape=jax.ShapeDtypeStruct((_N, _CH), _F32),
    )(acc_split, u, dinv, b_prev, W_next)


def _gelu(z):
    return z * 0.5 * (1.0 + jax.lax.erf(z * (2.0 ** -0.5)))


def _tc_final(acc_split, u, dinv, b_prev, batch2d,
              Wn1, bn1, gn1, ben1, Wn2, bn2, Wg1, bg1, gg1, beg1, Wg2, bg2):
    def body(acc_r, u_r, dinv_r, b_r, batch_r,
             Wn1_r, bn1_r, gn1_r, ben1_r, Wn2_r, bn2_r,
             Wg1_r, bg1_r, gg1_r, beg1_r, Wg2_r, bg2_r,
             node_ref, glob_ref):
        dinv = dinv_r[...]
        w = acc_r[0] + acc_r[1] + u_r[...]
        h = jnp.maximum(dinv * w + b_r[...], 0.0)
        z = _dot(h, Wn1_r[...]) + bn1_r[...]
        z = _bn_cols(z, gn1_r[...], ben1_r[...])
        node_ref[...] = _dot(_gelu(z), Wn2_r[...]) + bn2_r[...]
        gids = jax.lax.broadcasted_iota(jnp.int32, (_N, _G), 1)
        oh = (batch_r[...] == gids).astype(_F32)
        counts = jnp.sum(oh, axis=0, keepdims=True)
        sums = jax.lax.dot_general(
            oh, h, (((0,), (0,)), ((), ())),
            precision=jax.lax.Precision.HIGHEST, preferred_element_type=_F32)
        gf = sums / jnp.maximum(counts, 1.0).reshape(_G, 1)
        g = _dot(gf, Wg1_r[...]) + bg1_r[...]
        g = _bn_cols(g, gg1_r[...], beg1_r[...])
        glob_ref[...] = _dot(_gelu(g), Wg2_r[...]) + bg2_r[...]

    return pl.pallas_call(
        body,
        out_shape=[
            jax.ShapeDtypeStruct((_N, 2), _F32),
            jax.ShapeDtypeStruct((_G, 2), _F32),
        ],
    )(acc_split, u, dinv, b_prev, batch2d,
      Wn1, bn1, gn1, ben1, Wn2, bn2, Wg1, bg1, gg1, beg1, Wg2, bg2)


def kernel(x, edge_index, batch, lap_pe, params):
    p = params
    src_flat = edge_index[0].reshape(_NW, _CPT)
    dst32 = edge_index[1].reshape(_NW, _NCH, _CHUNK)
    ones_rows = jnp.ones((_CHUNK, _CH), _F32)
    zeros_pad = jnp.zeros((_ROWS_PT, _CH), _F32)
    row = lambda v: v.reshape(1, -1)

    degp = _sc_degree(dst32, ones_rows, zeros_pad)[:, :, :8]
    dinv = _tc_dinv(degp)
    u = _tc_prologue(
        x, lap_pe, dinv,
        p["W_node"], row(p["b_node"]), row(p["g_lap"]), row(p["be_lap"]),
        p["W_lap"], row(p["b_lap"]), p["convW"][0])
    for i in range(1, 3):
        acc = _sc_propagate(u, src_flat, dst32, zeros_pad)
        u = _tc_layer(acc, u, dinv, row(p["convb"][i - 1]), p["convW"][i])
    acc = _sc_propagate(u, src_flat, dst32, zeros_pad)
    node_pred, global_pred = _tc_final(
        acc, u, dinv, row(p["convb"][2]), batch.reshape(_N, 1),
        p["Wn1"], row(p["bn1"]), row(p["gn1"]), row(p["ben1"]),
        p["Wn2"], row(p["bn2"]),
        p["Wg1"], row(p["bg1"]), row(p["gg1"]), row(p["beg1"]),
        p["Wg2"], row(p["bg2"]))
    return node_pred, global_pred


# speedup vs baseline: 1.0135x; 1.0135x over previous
import functools

import jax
import jax.numpy as jnp
from jax import lax
from jax.experimental import pallas as pl
from jax.experimental.pallas import tpu as pltpu
from jax.experimental.pallas import tpu_sc as plsc

_N = 10000
_E = 320000
_CH = 128
_G = 16
_NC = 2
_NS = 16
_NW = _NC * _NS
_CHUNK = 80
_CPT = _E // _NW
_NCH = _CPT // _CHUNK
_NB = 2
_ROWS_PT = 624
_ROWS_REM = _N - _ROWS_PT * _NS

_F32 = jnp.float32


def _mesh():
    return plsc.VectorSubcoreMesh(core_axis_name="c", subcore_axis_name="s")


def _stripe_copy(src_of, dst_of, s):
    pltpu.sync_copy(src_of(s * _ROWS_PT, _ROWS_PT), dst_of(s * _ROWS_PT, _ROWS_PT))

    @pl.when(s == _NS - 1)
    def _tail():
        pltpu.sync_copy(src_of(_ROWS_PT * _NS, _ROWS_REM),
                        dst_of(_ROWS_PT * _NS, _ROWS_REM))


def _sc_degree(dst32, ones_rows, zeros_pad):
    @functools.partial(
        pl.kernel,
        out_type=jax.ShapeDtypeStruct((_NC, _N, _CH), _F32),
        mesh=_mesh(),
        scratch_types=[
            pltpu.VMEM((_NCH, _CHUNK), jnp.int32),
            pltpu.VMEM((_CHUNK, _CH), _F32),
            pltpu.VMEM_SHARED((_N, _CH), _F32),
        ],
    )
    def deg_kernel(dst_hbm, ones_hbm, z_hbm, out_hbm, idx_v, ones_v, acc_sh):
        c = lax.axis_index("c")
        s = lax.axis_index("s")

        _stripe_copy(lambda b, n: z_hbm.at[pl.ds(0, n)],
                     lambda b, n: acc_sh.at[pl.ds(b, n)], s)
        pltpu.sync_copy(dst_hbm.at[s * _NC + c], idx_v)
        pltpu.sync_copy(ones_hbm, ones_v)
        plsc.subcore_barrier()

        @pl.loop(0, _NCH)
        def _scat(j):
            pltpu.sync_copy(ones_v, acc_sh.at[idx_v.at[j]], add=True)

        plsc.subcore_barrier()
        _stripe_copy(lambda b, n: acc_sh.at[pl.ds(b, n)],
                     lambda b, n: out_hbm.at[c, pl.ds(b, n)], s)

    return deg_kernel(dst32, ones_rows, zeros_pad)


def _sc_propagate(u, src_flat, dst32, zeros_pad):
    @functools.partial(
        pl.kernel,
        out_type=jax.ShapeDtypeStruct((_NC, _N, _CH), _F32),
        mesh=_mesh(),
        scratch_types=[
            pltpu.VMEM((_CPT,), jnp.int32),
            pltpu.VMEM((_NCH, _CHUNK), jnp.int32),
        ] + [pltpu.VMEM((_CHUNK, _CH), _F32) for _ in range(_NB)] + [
            pltpu.VMEM_SHARED((_N, _CH), _F32),
        ] + [pltpu.SemaphoreType.DMA for _ in range(_NB)],
    )
    def prop_kernel(u_hbm, src_hbm, dst_hbm, z_hbm, out_hbm,
                    idx_s, idx_d, *rest):
        rows = rest[:_NB]
        acc_sh = rest[_NB]
        sems = rest[_NB + 1:]
        c = lax.axis_index("c")
        s = lax.axis_index("s")
        wid = s * _NC + c

        _stripe_copy(lambda b, n: z_hbm.at[pl.ds(0, n)],
                     lambda b, n: acc_sh.at[pl.ds(b, n)], s)
        pltpu.sync_copy(src_hbm.at[wid], idx_s)
        pltpu.sync_copy(dst_hbm.at[wid], idx_d)

        def _gather(j, b):
            pltpu.async_copy(u_hbm.at[idx_s.at[pl.ds(j * _CHUNK, _CHUNK)]],
                             rows[b], sems[b])

        def _wait(b):
            pltpu.make_async_copy(u_hbm.at[idx_s.at[pl.ds(0, _CHUNK)]],
                                  rows[b], sems[b]).wait()

        for b in range(_NB):
            _gather(b, b)
        plsc.subcore_barrier()

        @pl.loop(0, _NCH // _NB)
        def _ring(r):
            for b in range(_NB):
                j = r * _NB + b
                _wait(b)
                pltpu.sync_copy(rows[b], acc_sh.at[idx_d.at[j]], add=True)

                @pl.when(j + _NB < _NCH)
                def _reissue():
                    _gather(j + _NB, b)

        for t in range(_NCH % _NB):
            _wait(t)
            pltpu.sync_copy(rows[t],
                            acc_sh.at[idx_d.at[_NCH - _NCH % _NB + t]],
                            add=True)

        plsc.subcore_barrier()
        _stripe_copy(lambda b, n: acc_sh.at[pl.ds(b, n)],
                     lambda b, n: out_hbm.at[c, pl.ds(b, n)], s)

    return prop_kernel(u, src_flat, dst32, zeros_pad)


def _dot(a, b):
    return jax.lax.dot_general(
        a, b, (((1,), (0,)), ((), ())),
        precision=jax.lax.Precision.DEFAULT, preferred_element_type=_F32)


def _bn_cols(z, gamma, beta):
    mu = jnp.mean(z, axis=0, keepdims=True)
    var = jnp.mean((z - mu) * (z - mu), axis=0, keepdims=True)
    return (z - mu) * jax.lax.rsqrt(var + 1e-5) * gamma + beta


def _tc_prologue(x, lap_pe, degp, Wn, bn, gl, bel, Wl, bl, W0):
    def body(x_r, lap_r, degp_r, Wn_r, bn_r, gl_r, bel_r, Wl_r, bl_r, W0_r,
             u_ref, dinv_ref):
        deg = 1.0 + degp_r[0, :, 0:1] + degp_r[1, :, 0:1]
        dinv = jax.lax.rsqrt(deg)
        dinv_ref[...] = dinv
        xn = _dot(x_r[...], Wn_r[...]) + bn_r[...]
        lap = _bn_cols(lap_r[...], gl_r[...], bel_r[...])
        lap = _dot(lap, Wl_r[...]) + bl_r[...]
        h = jnp.concatenate([xn, lap], axis=1)
        u_ref[...] = _dot(h, W0_r[...]) * dinv

    return pl.pallas_call(
        body,
        out_shape=[
            jax.ShapeDtypeStruct((_N, _CH), _F32),
            jax.ShapeDtypeStruct((_N, 1), _F32),
        ],
    )(x, lap_pe, degp, Wn, bn, gl, bel, Wl, bl, W0)


def _tc_layer(acc_split, u, dinv, b_prev, W_next):
    def body(acc_r, u_r, dinv_r, b_r, W_r, out_ref):
        dinv = dinv_r[...]
        w = acc_r[0] + acc_r[1] + u_r[...]
        h = jnp.maximum(dinv * w + b_r[...], 0.0)
        out_ref[...] = _dot(h, W_r[...]) * dinv

    return pl.pallas_call(
        body,
        out_shape=jax.ShapeDtypeStruct((_N, _CH), _F32),
    )(acc_split, u, dinv, b_prev, W_next)


def _gelu(z):
    return z * 0.5 * (1.0 + jax.lax.erf(z * (2.0 ** -0.5)))


def _tc_final(acc_split, u, dinv, b_prev, batch2d,
              Wn1, bn1, gn1, ben1, Wn2, bn2, Wg1, bg1, gg1, beg1, Wg2, bg2):
    def body(acc_r, u_r, dinv_r, b_r, batch_r,
             Wn1_r, bn1_r, gn1_r, ben1_r, Wn2_r, bn2_r,
             Wg1_r, bg1_r, gg1_r, beg1_r, Wg2_r, bg2_r,
             node_ref, glob_ref):
        dinv = dinv_r[...]
        w = acc_r[0] + acc_r[1] + u_r[...]
        h = jnp.maximum(dinv * w + b_r[...], 0.0)
        z = _dot(h, Wn1_r[...]) + bn1_r[...]
        z = _bn_cols(z, gn1_r[...], ben1_r[...])
        node_ref[...] = _dot(_gelu(z), Wn2_r[...]) + bn2_r[...]
        gids = jax.lax.broadcasted_iota(jnp.int32, (_N, _G), 1)
        oh = (batch_r[...] == gids).astype(_F32)
        counts = jnp.sum(oh, axis=0, keepdims=True)
        sums = jax.lax.dot_general(
            oh, h, (((0,), (0,)), ((), ())),
            precision=jax.lax.Precision.HIGHEST, preferred_element_type=_F32)
        gf = sums / jnp.maximum(counts, 1.0).reshape(_G, 1)
        g = _dot(gf, Wg1_r[...]) + bg1_r[...]
        g = _bn_cols(g, gg1_r[...], beg1_r[...])
        glob_ref[...] = _dot(_gelu(g), Wg2_r[...]) + bg2_r[...]

    return pl.pallas_call(
        body,
        out_shape=[
            jax.ShapeDtypeStruct((_N, 2), _F32),
            jax.ShapeDtypeStruct((_G, 2), _F32),
        ],
    )(acc_split, u, dinv, b_prev, batch2d,
      Wn1, bn1, gn1, ben1, Wn2, bn2, Wg1, bg1, gg1, beg1, Wg2, bg2)


def kernel(x, edge_index, batch, lap_pe, params):
    p = params
    src_flat = edge_index[0].reshape(_NW, _CPT)
    dst32 = edge_index[1].reshape(_NW, _NCH, _CHUNK)
    ones_rows = jnp.ones((_CHUNK, _CH), _F32)
    zeros_pad = jnp.zeros((_ROWS_PT, _CH), _F32)
    row = lambda v: v.reshape(1, -1)

    degp = _sc_degree(dst32, ones_rows, zeros_pad)[:, :, :8]
    u, dinv = _tc_prologue(
        x, lap_pe, degp,
        p["W_node"], row(p["b_node"]), row(p["g_lap"]), row(p["be_lap"]),
        p["W_lap"], row(p["b_lap"]), p["convW"][0])
    for i in range(1, 3):
        acc = _sc_propagate(u, src_flat, dst32, zeros_pad)
        u = _tc_layer(acc, u, dinv, row(p["convb"][i - 1]), p["convW"][i])
    acc = _sc_propagate(u, src_flat, dst32, zeros_pad)
    node_pred, global_pred = _tc_final(
        acc, u, dinv, row(p["convb"][2]), batch.reshape(_N, 1),
        p["Wn1"], row(p["bn1"]), row(p["gn1"]), row(p["ben1"]),
        p["Wn2"], row(p["bn2"]),
        p["Wg1"], row(p["bg1"]), row(p["gg1"]), row(p["beg1"]),
        p["Wg2"], row(p["bg2"]))
    return node_pred, global_pred
